# Initial kernel scaffold; baseline (speedup 1.0000x reference)
import jax
import jax.numpy as jnp
from jax.experimental import pallas as pl


def kernel(categorical_inputs, numerical_inputs, tables, W1, b1, W2, b2):
    raise NotImplementedError("write your pallas kernel here")


# speedup vs baseline: 1.8746x; 1.8746x over previous
import jax
import jax.numpy as jnp
from jax.experimental import pallas as pl
from jax.experimental.pallas import tpu as pltpu
from jax.experimental.pallas import tpu_sc as plsc

_NUM_FIELDS = 26
_VOCAB = 100000
_EMBED_DIM = 32
_GATHER_WINDOW = 128


def _sc_gather(tables_flat, flat_idx):
    n = flat_idx.shape[1]
    d = tables_flat.shape[1]
    mesh = plsc.VectorSubcoreMesh(core_axis_name="core", subcore_axis_name="subcore")

    @pl.kernel(
        out_type=jax.ShapeDtypeStruct((n, d), jnp.float32),
        mesh=mesh,
        compiler_params=pltpu.CompilerParams(use_tc_tiling_on_sc=False),
    )
    def gather_kernel(tab_hbm, idx_hbm, out_hbm):
        def body(i_vmem, o_vmem):
            pltpu.sync_copy(tab_hbm.at[i_vmem.at[0]], o_vmem)

        pltpu.emit_pipeline(
            body,
            grid=(n // _GATHER_WINDOW,),
            in_specs=[pl.BlockSpec((1, _GATHER_WINDOW), index_map=lambda i: (0, i))],
            out_specs=[pl.BlockSpec((_GATHER_WINDOW, d), index_map=lambda i: (i, 0))],
            core_axis_name=("core", "subcore"),
            dimension_semantics=(pltpu.PARALLEL,),
        )(idx_hbm, out_hbm)

    return gather_kernel(tables_flat, flat_idx)


def _tc_mlp(emb, num, w1e, w1n, b1, w2t, b2):
    batch = emb.shape[0]
    blk = 1024
    in_dim = emb.shape[1]
    ndim = num.shape[1]
    hidden = w1e.shape[1]

    def body(emb_ref, num_ref, w1e_ref, w1n_ref, b1_ref, w2t_ref, b2_ref, out_ref):
        h = jax.lax.dot_general(
            emb_ref[...], w1e_ref[...], (((1,), (0,)), ((), ())),
            precision=jax.lax.Precision.HIGHEST,
            preferred_element_type=jnp.float32,
        )
        h = h + jax.lax.dot_general(
            num_ref[...], w1n_ref[...], (((1,), (0,)), ((), ())),
            precision=jax.lax.Precision.HIGHEST,
            preferred_element_type=jnp.float32,
        )
        h = jnp.maximum(h + b1_ref[...], 0.0)
        o = jnp.sum(h * w2t_ref[...], axis=1, keepdims=True) + b2_ref[...]
        out_ref[...] = jax.nn.sigmoid(o)

    return pl.pallas_call(
        body,
        grid=(batch // blk,),
        in_specs=[
            pl.BlockSpec((blk, in_dim), lambda i: (i, 0)),
            pl.BlockSpec((blk, ndim), lambda i: (i, 0)),
            pl.BlockSpec((in_dim, hidden), lambda i: (0, 0)),
            pl.BlockSpec((ndim, hidden), lambda i: (0, 0)),
            pl.BlockSpec((1, hidden), lambda i: (0, 0)),
            pl.BlockSpec((1, hidden), lambda i: (0, 0)),
            pl.BlockSpec((1, 1), lambda i: (0, 0)),
        ],
        out_specs=pl.BlockSpec((blk, 1), lambda i: (i, 0)),
        out_shape=jax.ShapeDtypeStruct((batch, 1), jnp.float32),
    )(emb, num, w1e, w1n, b1, w2t, b2)


def kernel(categorical_inputs, numerical_inputs, tables, W1, b1, W2, b2):
    batch, nf = categorical_inputs.shape
    d = tables.shape[2]
    tables_flat = tables.reshape(nf * _VOCAB, d)
    offsets = (jnp.arange(nf, dtype=jnp.int32) * _VOCAB)[None, :]
    flat_idx = (categorical_inputs + offsets).reshape(1, batch * nf)
    emb = _sc_gather(tables_flat, flat_idx).reshape(batch, nf * d)

    w1e = W1[: nf * d]
    w1n = W1[nf * d :]
    hidden = W1.shape[1]
    return _tc_mlp(
        emb,
        numerical_inputs,
        w1e,
        w1n,
        b1.reshape(1, hidden),
        W2.reshape(1, hidden),
        b2.reshape(1, 1),
    )


# speedup vs baseline: 6.4716x; 3.4522x over previous
import jax
import jax.numpy as jnp
from jax import lax
from jax.experimental import pallas as pl
from jax.experimental.pallas import tpu as pltpu
from jax.experimental.pallas import tpu_sc as plsc

_NUM_WORKERS = 32
_LANES = 16
_CHUNK = 4096


def _sc_gather_t(tables_t, cat_t, batch):
    nf, d, v = tables_t.shape
    rows = nf * d
    rpw = rows // _NUM_WORKERS
    n_chunks = batch // _CHUNK
    mesh = plsc.VectorSubcoreMesh(core_axis_name="core", subcore_axis_name="subcore")

    @pl.kernel(
        out_type=jax.ShapeDtypeStruct((rows, batch), jnp.float32),
        mesh=mesh,
        compiler_params=pltpu.CompilerParams(needs_layout_passes=False),
        scratch_types=[
            pltpu.VMEM((1, v), jnp.float32),
            pltpu.VMEM((1, _CHUNK), jnp.int32),
            pltpu.VMEM((1, _CHUNK), jnp.float32),
        ],
    )
    def gather_kernel(tab_hbm, cat_hbm, out_hbm, row_v, idx_v, val_v):
        wid = lax.axis_index("core") * 16 + lax.axis_index("subcore")
        zeros = jnp.zeros((_LANES,), jnp.int32)

        @pl.loop(0, rpw)
        def _row(k):
            r = wid * rpw + k
            f = r // d
            j = r - f * d
            pltpu.sync_copy(tab_hbm.at[f, j], row_v.at[0])

            @pl.loop(0, n_chunks)
            def _chunk(c):
                base = c * _CHUNK
                pltpu.sync_copy(cat_hbm.at[f, pl.ds(base, _CHUNK)], idx_v.at[0])

                @pl.loop(0, _CHUNK, step=8 * _LANES)
                def _blk(b0):
                    for u in range(8):
                        off = b0 + u * _LANES
                        idx = idx_v[0, pl.ds(off, _LANES)]
                        vals = plsc.load_gather(row_v, [zeros, idx])
                        val_v[0, pl.ds(off, _LANES)] = vals

                pltpu.sync_copy(val_v.at[0], out_hbm.at[r, pl.ds(base, _CHUNK)])

    return gather_kernel(tables_t, cat_t)


def _tc_mlp_t(emb_t, num_t, w1e_t, w1n_t, b1_col, w2_col, b2):
    rows, batch = emb_t.shape
    ndim = num_t.shape[0]
    hidden = w1e_t.shape[0]
    blk = 2048

    def body(embt_ref, numt_ref, w1et_ref, w1nt_ref, b1_ref, w2_ref, b2_ref, out_ref):
        ht = jax.lax.dot_general(
            w1et_ref[...], embt_ref[...], (((1,), (0,)), ((), ())),
            precision=jax.lax.Precision.HIGHEST,
            preferred_element_type=jnp.float32,
        )
        ht = ht + jax.lax.dot_general(
            w1nt_ref[...], numt_ref[...], (((1,), (0,)), ((), ())),
            precision=jax.lax.Precision.HIGHEST,
            preferred_element_type=jnp.float32,
        )
        ht = jnp.maximum(ht + b1_ref[...], 0.0)
        o = jnp.sum(ht * w2_ref[...], axis=0, keepdims=True) + b2_ref[...]
        out_ref[...] = jax.nn.sigmoid(o)

    return pl.pallas_call(
        body,
        grid=(batch // blk,),
        in_specs=[
            pl.BlockSpec((rows, blk), lambda i: (0, i)),
            pl.BlockSpec((ndim, blk), lambda i: (0, i)),
            pl.BlockSpec((hidden, rows), lambda i: (0, 0)),
            pl.BlockSpec((hidden, ndim), lambda i: (0, 0)),
            pl.BlockSpec((hidden, 1), lambda i: (0, 0)),
            pl.BlockSpec((hidden, 1), lambda i: (0, 0)),
            pl.BlockSpec((1, 1), lambda i: (0, 0)),
        ],
        out_specs=pl.BlockSpec((1, blk), lambda i: (0, i)),
        out_shape=jax.ShapeDtypeStruct((1, batch), jnp.float32),
    )(emb_t, num_t, w1e_t, w1n_t, b1_col, w2_col, b2)


def kernel(categorical_inputs, numerical_inputs, tables, W1, b1, W2, b2):
    batch, nf = categorical_inputs.shape
    d = tables.shape[2]
    hidden = W1.shape[1]

    tables_t = jnp.transpose(tables, (0, 2, 1))
    cat_t = jnp.transpose(categorical_inputs, (1, 0))
    num_t = jnp.transpose(numerical_inputs, (1, 0))

    emb_t = _sc_gather_t(tables_t, cat_t, batch)

    w1e_t = jnp.transpose(W1[: nf * d], (1, 0))
    w1n_t = jnp.transpose(W1[nf * d :], (1, 0))
    out_row = _tc_mlp_t(
        emb_t,
        num_t,
        w1e_t,
        w1n_t,
        b1.reshape(hidden, 1),
        W2.reshape(hidden, 1),
        b2.reshape(1, 1),
    )
    return out_row.reshape(batch, 1)


# speedup vs baseline: 10.9344x; 1.6896x over previous
import jax
import jax.numpy as jnp
from jax import lax
from jax.experimental import pallas as pl
from jax.experimental.pallas import tpu as pltpu
from jax.experimental.pallas import tpu_sc as plsc

_NUM_WORKERS = 32
_LANES = 16
_CHUNK = 4096


def _sc_gather_t(tables_t, cat_t, batch):
    nf, d, v = tables_t.shape
    rows = nf * d
    rpw = rows // _NUM_WORKERS
    n_chunks = batch // _CHUNK
    assert n_chunks % 2 == 0
    mesh = plsc.VectorSubcoreMesh(core_axis_name="core", subcore_axis_name="subcore")

    @pl.kernel(
        out_type=jax.ShapeDtypeStruct((rows, batch), jnp.float32),
        mesh=mesh,
        compiler_params=pltpu.CompilerParams(needs_layout_passes=False),
        scratch_types=[
            pltpu.VMEM((1, v), jnp.float32),
            pltpu.VMEM((1, batch), jnp.int32),
            pltpu.VMEM((2, _CHUNK), jnp.float32),
            pltpu.SemaphoreType.DMA,
            pltpu.SemaphoreType.DMA,
            pltpu.SemaphoreType.DMA,
        ],
    )
    def gather_kernel(tab_hbm, cat_hbm, out_hbm, row_v, idx_v, val_v, rsem, s0, s1):
        wid = lax.axis_index("core") * 16 + lax.axis_index("subcore")
        zeros = jnp.zeros((_LANES,), jnp.int32)
        ssems = (s0, s1)

        @pl.loop(0, rpw)
        def _row(k):
            r = wid * rpw + k
            f = r // d
            j = r - f * d
            row_cp = pltpu.async_copy(tab_hbm.at[f, j], row_v.at[0], rsem)

            @pl.when(k > 0)
            def _():
                for s in ssems:
                    pltpu.make_async_copy(
                        val_v.at[0], out_hbm.at[0, pl.ds(0, _CHUNK)], s
                    ).wait()

            @pl.when((k == 0) | (f != (r - 1) // d))
            def _():
                pltpu.sync_copy(cat_hbm.at[f], idx_v.at[0])

            row_cp.wait()

            for c in range(n_chunks):
                slot = c % 2
                base = c * _CHUNK
                if c >= 2:
                    pltpu.make_async_copy(
                        val_v.at[0], out_hbm.at[0, pl.ds(0, _CHUNK)], ssems[slot]
                    ).wait()

                @plsc.parallel_loop(0, _CHUNK, 16, unroll=8)
                def _blk(i):
                    idx = idx_v[0, pl.ds(base + i, _LANES)]
                    vals = plsc.load_gather(row_v, [zeros, idx])
                    val_v[slot, pl.ds(i, _LANES)] = vals

                pltpu.async_copy(
                    val_v.at[slot], out_hbm.at[r, pl.ds(base, _CHUNK)], ssems[slot]
                )

        for s in ssems:
            pltpu.make_async_copy(
                val_v.at[0], out_hbm.at[0, pl.ds(0, _CHUNK)], s
            ).wait()

    return gather_kernel(tables_t, cat_t)


def _tc_mlp_t(emb_t, num_t, w1e_t, w1n_t, b1_col, w2_col, b2):
    rows, batch = emb_t.shape
    ndim = num_t.shape[0]
    hidden = w1e_t.shape[0]
    blk = 2048

    def body(embt_ref, numt_ref, w1et_ref, w1nt_ref, b1_ref, w2_ref, b2_ref, out_ref):
        ht = jax.lax.dot_general(
            w1et_ref[...], embt_ref[...], (((1,), (0,)), ((), ())),
            precision=jax.lax.Precision.HIGHEST,
            preferred_element_type=jnp.float32,
        )
        ht = ht + jax.lax.dot_general(
            w1nt_ref[...], numt_ref[...], (((1,), (0,)), ((), ())),
            precision=jax.lax.Precision.HIGHEST,
            preferred_element_type=jnp.float32,
        )
        ht = jnp.maximum(ht + b1_ref[...], 0.0)
        o = jnp.sum(ht * w2_ref[...], axis=0, keepdims=True) + b2_ref[...]
        out_ref[...] = jax.nn.sigmoid(o)

    return pl.pallas_call(
        body,
        grid=(batch // blk,),
        in_specs=[
            pl.BlockSpec((rows, blk), lambda i: (0, i)),
            pl.BlockSpec((ndim, blk), lambda i: (0, i)),
            pl.BlockSpec((hidden, rows), lambda i: (0, 0)),
            pl.BlockSpec((hidden, ndim), lambda i: (0, 0)),
            pl.BlockSpec((hidden, 1), lambda i: (0, 0)),
            pl.BlockSpec((hidden, 1), lambda i: (0, 0)),
            pl.BlockSpec((1, 1), lambda i: (0, 0)),
        ],
        out_specs=pl.BlockSpec((1, blk), lambda i: (0, i)),
        out_shape=jax.ShapeDtypeStruct((1, batch), jnp.float32),
    )(emb_t, num_t, w1e_t, w1n_t, b1_col, w2_col, b2)


def kernel(categorical_inputs, numerical_inputs, tables, W1, b1, W2, b2):
    batch, nf = categorical_inputs.shape
    d = tables.shape[2]
    hidden = W1.shape[1]

    tables_t = jnp.transpose(tables, (0, 2, 1))
    cat_t = jnp.transpose(categorical_inputs, (1, 0))
    num_t = jnp.transpose(numerical_inputs, (1, 0))

    emb_t = _sc_gather_t(tables_t, cat_t, batch)

    w1e_t = jnp.transpose(W1[: nf * d], (1, 0))
    w1n_t = jnp.transpose(W1[nf * d :], (1, 0))
    out_row = _tc_mlp_t(
        emb_t,
        num_t,
        w1e_t,
        w1n_t,
        b1.reshape(hidden, 1),
        W2.reshape(hidden, 1),
        b2.reshape(1, 1),
    )
    return out_row.reshape(batch, 1)


# speedup vs baseline: 11.4587x; 1.0479x over previous
import jax
import jax.numpy as jnp
from jax import lax
from jax.experimental import pallas as pl
from jax.experimental.pallas import tpu as pltpu
from jax.experimental.pallas import tpu_sc as plsc

_NUM_WORKERS = 32
_LANES = 16
_CHUNK = 4096


def _sc_gather_t(tables_t, cat_t, batch, row_lo, n_rows):
    nf, d, v = tables_t.shape
    rpw = n_rows // _NUM_WORKERS
    n_chunks = batch // _CHUNK
    assert n_chunks % 2 == 0
    mesh = plsc.VectorSubcoreMesh(core_axis_name="core", subcore_axis_name="subcore")

    @pl.kernel(
        out_type=jax.ShapeDtypeStruct((n_rows, batch), jnp.float32),
        mesh=mesh,
        compiler_params=pltpu.CompilerParams(needs_layout_passes=False),
        scratch_types=[
            pltpu.VMEM((1, v), jnp.float32),
            pltpu.VMEM((1, batch), jnp.int32),
            pltpu.VMEM((2, _CHUNK), jnp.float32),
            pltpu.SemaphoreType.DMA,
            pltpu.SemaphoreType.DMA,
            pltpu.SemaphoreType.DMA,
        ],
    )
    def gather_kernel(tab_hbm, cat_hbm, out_hbm, row_v, idx_v, val_v, rsem, s0, s1):
        wid = lax.axis_index("core") * 16 + lax.axis_index("subcore")
        zeros = jnp.zeros((_LANES,), jnp.int32)
        ssems = (s0, s1)

        @pl.loop(0, rpw)
        def _row(k):
            lr = wid * rpw + k
            r = row_lo + lr
            f = r // d
            j = r - f * d
            row_cp = pltpu.async_copy(tab_hbm.at[f, j], row_v.at[0], rsem)

            @pl.when(k > 0)
            def _():
                for s in ssems:
                    pltpu.make_async_copy(
                        val_v.at[0], out_hbm.at[0, pl.ds(0, _CHUNK)], s
                    ).wait()

            @pl.when((k == 0) | (f != (r - 1) // d))
            def _():
                pltpu.sync_copy(cat_hbm.at[f], idx_v.at[0])

            row_cp.wait()

            for c in range(n_chunks):
                slot = c % 2
                base = c * _CHUNK
                if c >= 2:
                    pltpu.make_async_copy(
                        val_v.at[0], out_hbm.at[0, pl.ds(0, _CHUNK)], ssems[slot]
                    ).wait()

                @plsc.parallel_loop(0, _CHUNK, 16, unroll=8)
                def _blk(i):
                    idx = idx_v[0, pl.ds(base + i, _LANES)]
                    vals = plsc.load_gather(row_v, [zeros, idx])
                    val_v[slot, pl.ds(i, _LANES)] = vals

                pltpu.async_copy(
                    val_v.at[slot], out_hbm.at[lr, pl.ds(base, _CHUNK)], ssems[slot]
                )

        for s in ssems:
            pltpu.make_async_copy(
                val_v.at[0], out_hbm.at[0, pl.ds(0, _CHUNK)], s
            ).wait()

    return gather_kernel(tables_t, cat_t)


_BLK = 2048


def _tc_partial(w_t, emb_t, num_t, w1n_t):
    hidden, rows = w_t.shape
    batch = emb_t.shape[1]
    ndim = num_t.shape[0]

    def body(w_ref, e_ref, n_ref, wn_ref, o_ref):
        o_ref[...] = jax.lax.dot_general(
            w_ref[...], e_ref[...], (((1,), (0,)), ((), ())),
            precision=jax.lax.Precision.DEFAULT,
            preferred_element_type=jnp.float32,
        ) + jax.lax.dot_general(
            wn_ref[...], n_ref[...], (((1,), (0,)), ((), ())),
            precision=jax.lax.Precision.DEFAULT,
            preferred_element_type=jnp.float32,
        )

    return pl.pallas_call(
        body,
        grid=(batch // _BLK,),
        in_specs=[
            pl.BlockSpec((hidden, rows), lambda i: (0, 0)),
            pl.BlockSpec((rows, _BLK), lambda i: (0, i)),
            pl.BlockSpec((ndim, _BLK), lambda i: (0, i)),
            pl.BlockSpec((hidden, ndim), lambda i: (0, 0)),
        ],
        out_specs=pl.BlockSpec((hidden, _BLK), lambda i: (0, i)),
        out_shape=jax.ShapeDtypeStruct((hidden, batch), jnp.float32),
    )(w_t, emb_t, num_t, w1n_t)


def _tc_final(partial, emb_t, w_t, b1_col, w2_col, b2):
    hidden, rows = w_t.shape
    batch = emb_t.shape[1]

    def body(p_ref, e_ref, w_ref, b1_ref, w2_ref, b2_ref, out_ref):
        ht = p_ref[...] + jax.lax.dot_general(
            w_ref[...], e_ref[...], (((1,), (0,)), ((), ())),
            precision=jax.lax.Precision.DEFAULT,
            preferred_element_type=jnp.float32,
        )
        ht = jnp.maximum(ht + b1_ref[...], 0.0)
        o = jnp.sum(ht * w2_ref[...], axis=0, keepdims=True) + b2_ref[...]
        out_ref[...] = jax.nn.sigmoid(o)

    return pl.pallas_call(
        body,
        grid=(batch // _BLK,),
        in_specs=[
            pl.BlockSpec((hidden, _BLK), lambda i: (0, i)),
            pl.BlockSpec((rows, _BLK), lambda i: (0, i)),
            pl.BlockSpec((hidden, rows), lambda i: (0, 0)),
            pl.BlockSpec((hidden, 1), lambda i: (0, 0)),
            pl.BlockSpec((hidden, 1), lambda i: (0, 0)),
            pl.BlockSpec((1, 1), lambda i: (0, 0)),
        ],
        out_specs=pl.BlockSpec((1, _BLK), lambda i: (0, i)),
        out_shape=jax.ShapeDtypeStruct((1, batch), jnp.float32),
    )(partial, emb_t, w_t, b1_col, w2_col, b2)


def kernel(categorical_inputs, numerical_inputs, tables, W1, b1, W2, b2):
    batch, nf = categorical_inputs.shape
    d = tables.shape[2]
    hidden = W1.shape[1]

    tables_t = jnp.transpose(tables, (0, 2, 1))
    cat_t = jnp.transpose(categorical_inputs, (1, 0))
    num_t = jnp.transpose(numerical_inputs, (1, 0))

    rows = nf * d
    half = (nf // 2) * d

    emb0 = _sc_gather_t(tables_t, cat_t, batch, 0, half)
    emb1 = _sc_gather_t(tables_t, cat_t, batch, half, rows - half)

    w1e_t = jnp.transpose(W1[:rows], (1, 0))
    w1n_t = jnp.transpose(W1[rows:], (1, 0))
    partial = _tc_partial(w1e_t[:, :half], emb0, num_t, w1n_t)
    out_row = _tc_final(
        partial,
        emb1,
        w1e_t[:, half:],
        b1.reshape(hidden, 1),
        W2.reshape(hidden, 1),
        b2.reshape(1, 1),
    )
    return out_row.reshape(batch, 1)
